# baseline (speedup 1.0000x reference)
import functools

import jax
import jax.numpy as jnp
from jax import lax
from jax.experimental import pallas as pl
from jax.experimental.pallas import tpu as pltpu
from jax.experimental.pallas import tpu_sc as plsc

NROW = 100
EMBED_DIM = 768
LANES = 16
NC, NS = 2, 16
NW = NC * NS
B_TOTAL = 4096 * 50
B_PER_W = B_TOTAL // NW
CHUNK = 80
NGROUP = CHUNK // LANES
NCHUNK = B_PER_W // CHUNK
NPAIR = NCHUNK // 2
ROW = EMBED_DIM


def _emb_body(ids_hbm, table_hbm, out_hbm, idxbuf, ob0, ob1, table_v,
              i0, i1, o0, o1, r0, r1):
    wid = lax.axis_index("s") * NC + lax.axis_index("c")
    base = wid * B_PER_W

    pltpu.sync_copy(table_hbm, table_v)

    obufs = (ob0, ob1)
    isem = (i0, i1)
    osem = (o0, o1)
    rsem = (r0, r1)

    def idx_desc(g, b):
        return pltpu.make_async_copy(
            ids_hbm.at[pl.ds(base + g * CHUNK, CHUNK)], idxbuf.at[b], isem[b])

    def out_desc(g, b):
        return pltpu.make_async_copy(
            obufs[b],
            out_hbm.at[pl.ds((base + g * CHUNK) * ROW, CHUNK * ROW)],
            osem[b])

    def compute(b, rsem):
        ob = obufs[b]
        for j in range(NGROUP):
            ids16 = idxbuf[b, pl.ds(j * LANES, LANES)]
            for r in range(LANES):
                bid = ids16[r]
                pltpu.make_async_copy(
                    table_v.at[pl.ds(bid * ROW, ROW)],
                    ob.at[pl.ds((j * LANES + r) * ROW, ROW)],
                    rsem).start()

    def compute_wait(b, rsem):
        ob = obufs[b]
        for rr in range(CHUNK):
            pltpu.make_async_copy(
                table_v.at[pl.ds(0, ROW)],
                ob.at[pl.ds(rr * ROW, ROW)], rsem).wait()

    for b in range(2):
        idx_desc(b, b).start()
    for b in range(2):
        idx_desc(b, b).wait()
        compute(b, rsem[b])
        compute_wait(b, rsem[b])
        out_desc(b, b).start()
        idx_desc(b + 2, b).start()

    def body(i, carry):
        g = 2 * i
        for b in range(2):
            gg = g + b
            idx_desc(gg, b).wait()
            out_desc(gg - 2, b).wait()
            compute(b, rsem[b])
            compute_wait(b, rsem[b])
            out_desc(gg, b).start()
            idx_desc(gg + 2, b).start()
        return carry

    lax.fori_loop(1, NPAIR - 1, body, 0)

    for b in range(2):
        gg = NCHUNK - 2 + b
        idx_desc(gg, b).wait()
        out_desc(gg - 2, b).wait()
        compute(b, rsem[b])
        compute_wait(b, rsem[b])
        out_desc(gg, b).start()
    for b in range(2):
        out_desc(NCHUNK - 2 + b, b).wait()


@functools.partial(jax.jit, static_argnums=())
def _emb_lookup(ids_flat, table):
    mesh = plsc.VectorSubcoreMesh(core_axis_name="c", subcore_axis_name="s")
    f = pl.kernel(
        _emb_body,
        out_type=jax.ShapeDtypeStruct((B_TOTAL * EMBED_DIM,), jnp.float32),
        mesh=mesh,
        compiler_params=pltpu.CompilerParams(needs_layout_passes=False),
        scratch_types=[
            pltpu.VMEM((2, CHUNK), jnp.int32),
            pltpu.VMEM((CHUNK * ROW,), jnp.float32),
            pltpu.VMEM((CHUNK * ROW,), jnp.float32),
            pltpu.VMEM_SHARED((NROW * ROW,), jnp.float32),
            pltpu.SemaphoreType.DMA,
            pltpu.SemaphoreType.DMA,
            pltpu.SemaphoreType.DMA,
            pltpu.SemaphoreType.DMA,
            pltpu.SemaphoreType.DMA,
            pltpu.SemaphoreType.DMA,
        ],
    )
    return f(ids_flat, table.reshape(-1))


def kernel(input_ids, embedding_weight):
    ids = input_ids.reshape(-1)
    out = _emb_lookup(ids, embedding_weight)
    return out.reshape(input_ids.shape + (EMBED_DIM,))


# speedup vs baseline: 1.0011x; 1.0011x over previous
import functools

import jax
import jax.numpy as jnp
from jax import lax
from jax.experimental import pallas as pl
from jax.experimental.pallas import tpu as pltpu
from jax.experimental.pallas import tpu_sc as plsc

NROW = 100
EMBED_DIM = 768
LANES = 16
NC, NS = 2, 16
NW = NC * NS
B_TOTAL = 4096 * 50
B_PER_W = B_TOTAL // NW
CHUNK = 80
NGROUP = CHUNK // LANES
NCHUNK = B_PER_W // CHUNK
NPAIR = NCHUNK // 2
ROW = EMBED_DIM


def _emb_body(ids_hbm, table_hbm, out_hbm, idxbuf, ob0, ob1, table_v,
              i0, i1, o0, o1, r0, r1):
    wid = lax.axis_index("s") * NC + lax.axis_index("c")
    base = wid * B_PER_W

    pltpu.sync_copy(table_hbm, table_v)

    obufs = (ob0, ob1)
    isem = (i0, i1)
    osem = (o0, o1)
    rsem = (r0, r1)

    def idx_desc(g, b):
        return pltpu.make_async_copy(
            ids_hbm.at[pl.ds(base + g * CHUNK, CHUNK)], idxbuf.at[b], isem[b])

    def out_desc(g, b):
        return pltpu.make_async_copy(
            obufs[b],
            out_hbm.at[pl.ds((base + g * CHUNK) * ROW, CHUNK * ROW)],
            osem[b])

    def compute(b, rsem):
        ob = obufs[b]
        for j in range(NGROUP):
            ids16 = idxbuf[b, pl.ds(j * LANES, LANES)]
            for r in range(LANES):
                bid = ids16[r]
                pltpu.make_async_copy(
                    table_v.at[pl.ds(bid * ROW, ROW)],
                    ob.at[pl.ds((j * LANES + r) * ROW, ROW)],
                    rsem).start()

    def compute_wait(b, rsem):
        pltpu.make_async_copy(
            table_v.at[pl.ds(0, CHUNK * ROW)], obufs[b], rsem).wait()

    for b in range(2):
        idx_desc(b, b).start()
    for b in range(2):
        idx_desc(b, b).wait()
        compute(b, rsem[b])
        compute_wait(b, rsem[b])
        out_desc(b, b).start()
        idx_desc(b + 2, b).start()

    def body(i, carry):
        g = 2 * i
        for b in range(2):
            gg = g + b
            idx_desc(gg, b).wait()
            out_desc(gg - 2, b).wait()
            compute(b, rsem[b])
            compute_wait(b, rsem[b])
            out_desc(gg, b).start()
            idx_desc(gg + 2, b).start()
        return carry

    lax.fori_loop(1, NPAIR - 1, body, 0)

    for b in range(2):
        gg = NCHUNK - 2 + b
        idx_desc(gg, b).wait()
        out_desc(gg - 2, b).wait()
        compute(b, rsem[b])
        compute_wait(b, rsem[b])
        out_desc(gg, b).start()
    for b in range(2):
        out_desc(NCHUNK - 2 + b, b).wait()


@functools.partial(jax.jit, static_argnums=())
def _emb_lookup(ids_flat, table):
    mesh = plsc.VectorSubcoreMesh(core_axis_name="c", subcore_axis_name="s")
    f = pl.kernel(
        _emb_body,
        out_type=jax.ShapeDtypeStruct((B_TOTAL * EMBED_DIM,), jnp.float32),
        mesh=mesh,
        compiler_params=pltpu.CompilerParams(needs_layout_passes=False),
        scratch_types=[
            pltpu.VMEM((2, CHUNK), jnp.int32),
            pltpu.VMEM((CHUNK * ROW,), jnp.float32),
            pltpu.VMEM((CHUNK * ROW,), jnp.float32),
            pltpu.VMEM_SHARED((NROW * ROW,), jnp.float32),
            pltpu.SemaphoreType.DMA,
            pltpu.SemaphoreType.DMA,
            pltpu.SemaphoreType.DMA,
            pltpu.SemaphoreType.DMA,
            pltpu.SemaphoreType.DMA,
            pltpu.SemaphoreType.DMA,
        ],
    )
    return f(ids_flat, table.reshape(-1))


def kernel(input_ids, embedding_weight):
    ids = input_ids.reshape(-1)
    out = _emb_lookup(ids, embedding_weight)
    return out.reshape(input_ids.shape + (EMBED_DIM,))
